# baseline (speedup 1.0000x reference)
import functools

import jax
import jax.numpy as jnp
from jax import lax
from jax.experimental import pallas as pl
from jax.experimental.pallas import tpu as pltpu
from jax.experimental.pallas import tpu_sc as plsc

N_FIELDS = 26
VOCAB = 100000
DIM = 16

NC = 2
NS = 16
NW = NC * NS
LANES = 16


def _make_kernel(batch):
    b_per_w = batch // NW
    chunk = 128
    n_chunks = b_per_w // chunk
    w = chunk * N_FIELDS

    mesh = plsc.VectorSubcoreMesh(core_axis_name="c", subcore_axis_name="s")

    @functools.partial(
        pl.kernel,
        mesh=mesh,
        compiler_params=pltpu.CompilerParams(use_tc_tiling_on_sc=False),
        out_type=[
            jax.ShapeDtypeStruct((batch * N_FIELDS, DIM), jnp.float32),
            jax.ShapeDtypeStruct((batch,), jnp.float32),
        ],
        scratch_types=[
            pltpu.VMEM((w,), jnp.int32),
            pltpu.VMEM((w,), jnp.int32),
            pltpu.VMEM((w,), jnp.int32),
            pltpu.VMEM((w,), jnp.int32),
            pltpu.VMEM((w,), jnp.int32),
            pltpu.VMEM((w,), jnp.int32),
            pltpu.VMEM((w, DIM), jnp.float32),
            pltpu.VMEM((w,), jnp.float32),
            pltpu.VMEM((chunk,), jnp.float32),
            pltpu.SemaphoreType.DMA,
            pltpu.SemaphoreType.DMA,
        ],
    )
    def k(x_hbm, xt_hbm, offs_hbm, offst_hbm, emb_hbm, lin_hbm,
          dnn_hbm, logit_hbm,
          x_v, xt_v, idx_v, idxt_v, offs_v, offst_v, emb_v, lin_v, logit_v,
          sem_e, sem_l):
        wid = lax.axis_index("s") * NC + lax.axis_index("c")
        pltpu.sync_copy(offs_hbm, offs_v)
        pltpu.sync_copy(offst_hbm, offst_v)

        def chunk_body(c, carry):
            base_b = wid * b_per_w + c * chunk
            base_i = base_b * N_FIELDS
            pltpu.sync_copy(x_hbm.at[pl.ds(base_i, w)], x_v)
            pltpu.sync_copy(xt_hbm.at[pl.ds(base_i, w)], xt_v)

            def idx_body(i, carry2):
                s = pl.ds(i * LANES, LANES)
                idx_v[s] = x_v[s] + offs_v[s]
                idxt_v[s] = xt_v[s] + offst_v[s]
                return carry2

            lax.fori_loop(0, w // LANES, idx_body, 0)

            cp_e = pltpu.async_copy(emb_hbm.at[idx_v], emb_v, sem_e)
            cp_l = pltpu.async_copy(lin_hbm.at[idxt_v], lin_v, sem_l)
            cp_l.wait()

            def logit_body(g, carry2):
                acc = lin_v[pl.ds(g * LANES, LANES)]
                for j in range(1, N_FIELDS):
                    acc = acc + lin_v[pl.ds(j * chunk + g * LANES, LANES)]
                logit_v[pl.ds(g * LANES, LANES)] = acc
                return carry2

            lax.fori_loop(0, chunk // LANES, logit_body, 0)

            cp_e.wait()
            pltpu.sync_copy(emb_v, dnn_hbm.at[pl.ds(base_i, w)])
            pltpu.sync_copy(logit_v, logit_hbm.at[pl.ds(base_b, chunk)])
            return carry

        lax.fori_loop(0, n_chunks, chunk_body, 0)

    return k


def kernel(x, emb_tables, lin_tables):
    batch = x.shape[0]
    chunk = 128
    emb_flat = emb_tables.reshape(N_FIELDS * VOCAB, DIM)
    lin_flat = lin_tables.reshape(N_FIELDS * VOCAB)
    x_flat = x.reshape(-1)
    xt_flat = x.reshape(batch // chunk, chunk, N_FIELDS)
    xt_flat = xt_flat.transpose(0, 2, 1).reshape(-1)
    offs = jnp.tile(jnp.arange(N_FIELDS, dtype=jnp.int32) * VOCAB, chunk)
    offst = jnp.repeat(jnp.arange(N_FIELDS, dtype=jnp.int32) * VOCAB, chunk)
    dnn, logit = _make_kernel(batch)(
        x_flat, xt_flat, offs, offst, emb_flat, lin_flat)
    return jnp.concatenate(
        [dnn.reshape(batch, N_FIELDS * DIM), logit.reshape(batch, 1)], axis=1
    )


# speedup vs baseline: 1.0043x; 1.0043x over previous
import functools

import jax
import jax.numpy as jnp
from jax import lax
from jax.experimental import pallas as pl
from jax.experimental.pallas import tpu as pltpu
from jax.experimental.pallas import tpu_sc as plsc

N_FIELDS = 26
VOCAB = 100000
DIM = 16
OUT_D = N_FIELDS * DIM + 1

NC = 2
NS = 16
NW = NC * NS
LANES = 16

CHUNK = 128


def _make_kernel(batch):
    b_per_w = batch // NW
    chunk = CHUNK
    n_chunks = b_per_w // chunk
    w = chunk * N_FIELDS
    ids_per_w = b_per_w * N_FIELDS

    mesh = plsc.VectorSubcoreMesh(core_axis_name="c", subcore_axis_name="s")

    @functools.partial(
        pl.kernel,
        mesh=mesh,
        compiler_params=pltpu.CompilerParams(use_tc_tiling_on_sc=False),
        out_type=[
            jax.ShapeDtypeStruct((batch, N_FIELDS * DIM), jnp.float32),
            jax.ShapeDtypeStruct((batch,), jnp.float32),
        ],
        scratch_types=[
            pltpu.VMEM((ids_per_w,), jnp.int32),
            pltpu.VMEM((2, w, DIM), jnp.float32),
            pltpu.VMEM((2, w), jnp.float32),
            pltpu.VMEM((2, chunk), jnp.float32),
            pltpu.SemaphoreType.DMA,
            pltpu.SemaphoreType.DMA,
            pltpu.SemaphoreType.DMA,
            pltpu.SemaphoreType.DMA,
            pltpu.SemaphoreType.DMA,
            pltpu.SemaphoreType.DMA,
        ],
    )
    def k(idxf_hbm, emb_hbm, lin_hbm, out_hbm, outl_hbm,
          idxf_v, emb_v, lin_v, logit_v,
          sem_e0, sem_e1, sem_l0, sem_l1, sem_w0, sem_w1):
        wid = lax.axis_index("s") * NC + lax.axis_index("c")
        base_i = wid * ids_per_w
        pltpu.sync_copy(idxf_hbm.at[pl.ds(base_i, ids_per_w)], idxf_v)

        sem_e = (sem_e0, sem_e1)
        sem_l = (sem_l0, sem_l1)
        sem_w = (sem_w0, sem_w1)

        def start_gathers(c, s):
            cp_e = pltpu.async_copy(
                emb_hbm.at[idxf_v.at[pl.ds(c * w, w)]],
                emb_v.at[s], sem_e[s])
            cp_l = pltpu.async_copy(
                lin_hbm.at[idxf_v.at[pl.ds(c * w, w)]],
                lin_v.at[s], sem_l[s])
            return cp_e, cp_l

        pend_w = [None, None]
        pend_g = [None, None]
        pend_g[0] = start_gathers(0, 0)

        for c in range(n_chunks):
            s = c % 2
            o = 1 - s
            if c + 1 < n_chunks:
                if pend_w[o] is not None:
                    for h in pend_w[o]:
                        h.wait()
                    pend_w[o] = None
                pend_g[o] = start_gathers(c + 1, o)

            base_b = wid * b_per_w + c * chunk
            cp_e, cp_l = pend_g[s]
            cp_l.wait()

            def logit_body(g, carry2):
                acc = lin_v[s, pl.ds(g * LANES, LANES)]
                for j in range(1, N_FIELDS):
                    acc = acc + lin_v[s, pl.ds(j * chunk + g * LANES, LANES)]
                logit_v[s, pl.ds(g * LANES, LANES)] = acc
                return carry2

            lax.fori_loop(0, chunk // LANES, logit_body, 0)

            cp_e.wait()
            writes = []
            for f in range(N_FIELDS):
                writes.append(pltpu.async_copy(
                    emb_v.at[s, pl.ds(f * chunk, chunk)],
                    out_hbm.at[pl.ds(base_b, chunk), pl.ds(f * DIM, DIM)],
                    sem_w[s]))
            writes.append(pltpu.async_copy(
                logit_v.at[s],
                outl_hbm.at[pl.ds(base_b, chunk)],
                sem_w[s]))
            pend_w[s] = writes

        for pw in pend_w:
            if pw is not None:
                for h in pw:
                    h.wait()

    return k


def kernel(x, emb_tables, lin_tables):
    batch = x.shape[0]
    emb_flat = emb_tables.reshape(N_FIELDS * VOCAB, DIM)
    lin_flat = lin_tables.reshape(N_FIELDS * VOCAB)
    offs = jnp.arange(N_FIELDS, dtype=jnp.int32) * VOCAB
    idx = x + offs[None, :]
    idxf = idx.reshape(batch // CHUNK, CHUNK, N_FIELDS)
    idxf = idxf.transpose(0, 2, 1).reshape(-1)
    out_emb, out_logit = _make_kernel(batch)(idxf, emb_flat, lin_flat)
    return jnp.concatenate([out_emb, out_logit[:, None]], axis=1)
